# baseline (speedup 1.0000x reference)
import functools

import jax
import jax.numpy as jnp
from jax import lax
from jax.experimental import pallas as pl
from jax.experimental.pallas import tpu as pltpu
from jax.experimental.pallas import tpu_sc as plsc

B = 4
N1 = 4096
N2 = 4096
K = 16
NSEL = K + 1
QPW = 512
NCH = N2 // 16
WARM = 4
BNDS = (4, 20, 256)

_INF = float(jnp.inf)


def _lanes():
    return lax.broadcasted_iota(jnp.int32, (16,), 0)


def _last15():
    return jnp.full((16,), 15, jnp.int32)


def _vmaxs(v):
    return jnp.take_along_axis(plsc.cummax(v), _last15(), axis=0)


def _vmins(v):
    return -_vmaxs(-v)


def _bound17v(m1, m2):
    a = lax.sort(m1)
    bb = lax.rev(lax.sort(m2), (0,))
    return _vmins(jnp.maximum(a, bb))


def _sc_body(x2x, x2y, x2z, x2xr, x2yr, x2zr,
             x1x, x1y, x1z, x1xr, x1yr, x1zr, out,
             rx, ry, rz, rn, ux, uy, uz,
             qx, qy, qz, qn, vx, vy, vz,
             sd, si, sdB, siB, s2d, s2i, s2dB, s2iB, ob):
    cid = lax.axis_index("c")
    sid = lax.axis_index("s")
    wid = sid * 2 + cid
    b = wid // (N1 // QPW)
    qs = (wid % (N1 // QPW)) * QPW

    pltpu.sync_copy(x2x.at[b], ux)
    pltpu.sync_copy(x2y.at[b], uy)
    pltpu.sync_copy(x2z.at[b], uz)
    pltpu.sync_copy(x2xr.at[b], rx)
    pltpu.sync_copy(x2yr.at[b], ry)
    pltpu.sync_copy(x2zr.at[b], rz)
    pltpu.sync_copy(x1x.at[b, pl.ds(qs, QPW)], vx)
    pltpu.sync_copy(x1y.at[b, pl.ds(qs, QPW)], vy)
    pltpu.sync_copy(x1z.at[b, pl.ds(qs, QPW)], vz)
    pltpu.sync_copy(x1xr.at[b, pl.ds(qs, QPW)], qx)
    pltpu.sync_copy(x1yr.at[b, pl.ds(qs, QPW)], qy)
    pltpu.sync_copy(x1zr.at[b, pl.ds(qs, QPW)], qz)

    lanes = _lanes()

    def ref_pre(i, _):
        s = pl.ds(i * 16, 16)
        rn[s] = (ux[s] * ux[s] + uy[s] * uy[s]) + uz[s] * uz[s]
        return 0

    lax.fori_loop(0, NCH, ref_pre, 0, unroll=4)

    def q_pre(i, _):
        s = pl.ds(i * 16, 16)
        qn[s] = (vx[s] * vx[s] + vy[s] * vy[s]) + vz[s] * vz[s]
        return 0

    lax.fori_loop(0, QPW // 16, q_pre, 0, unroll=4)

    def idx_pre(i, _):
        s = pl.ds(i * 16, 16)
        iv = i * 16 + lanes
        si[s] = iv
        siB[s] = iv
        return 0

    lax.fori_loop(0, WARM, idx_pre, 0, unroll=4)

    def per_pair(pq, _):
        def qload(ref, qsplat):
            return plsc.load_gather(ref, [qsplat])

        qa = jnp.broadcast_to(pq * 2, (16,)).astype(jnp.int32)
        qb = qa + 1
        qxa, qya, qza, qna = (qload(qx, qa), qload(qy, qa),
                              qload(qz, qa), qload(qn, qa))
        qxb, qyb, qzb, qnb = (qload(qx, qb), qload(qy, qb),
                              qload(qz, qb), qload(qn, qb))

        def dist2(i):
            s = pl.ds(i * 16, 16)
            vrx, vry, vrz, vrn = rx[s], ry[s], rz[s], rn[s]

            def one(qxv, qyv, qzv, qnv):
                cross = vrx * qxv + vry * qyv + vrz * qzv
                d = (vrn + qnv) - (cross + cross)
                return d, jnp.maximum(d, jnp.float32(0.0))

            return one(qxa, qya, qza, qna), one(qxb, qyb, qzb, qnb)

        def minmax2(dc, m1, m2):
            hi = jnp.maximum(m1, dc)
            return jnp.minimum(m1, dc), jnp.minimum(m2, hi)

        def warm(i, carry):
            m1a, m2a, m1b, m2b = carry
            (_, dca), (_, dcb) = dist2(i)
            s = pl.ds(i * 16, 16)
            sd[s] = dca
            sdB[s] = dcb
            m1a, m2a = minmax2(dca, m1a, m2a)
            m1b, m2b = minmax2(dcb, m1b, m2b)
            return m1a, m2a, m1b, m2b

        inf16 = jnp.full((16,), _INF)
        m1a, m2a, m1b, m2b = plsc.parallel_loop(
            0, WARM, carry=(inf16, inf16, inf16, inf16), unroll=4)(warm)
        cnt0 = jnp.full((16,), WARM * 16, jnp.int32)

        def chunk(ta, tb):
            def go(i, carry):
                cnta, m1a, m2a, cntb, m1b, m2b = carry
                (da, dca), (db, dcb) = dist2(i)
                idxv = i * 16 + lanes
                mska = da <= ta
                posa = cnta + plsc.cumsum(mska.astype(jnp.int32)) - 1
                plsc.store_scatter(sd, [posa], dca, mask=mska)
                plsc.store_scatter(si, [posa], idxv, mask=mska)
                cnta = cnta + plsc.all_reduce_population_count(mska)
                m1a, m2a = minmax2(dca, m1a, m2a)
                mskb = db <= tb
                posb = cntb + plsc.cumsum(mskb.astype(jnp.int32)) - 1
                plsc.store_scatter(sdB, [posb], dcb, mask=mskb)
                plsc.store_scatter(siB, [posb], idxv, mask=mskb)
                cntb = cntb + plsc.all_reduce_population_count(mskb)
                m1b, m2b = minmax2(dcb, m1b, m2b)
                return cnta, m1a, m2a, cntb, m1b, m2b
            return go

        carry = (cnt0, m1a, m2a, cnt0, m1b, m2b)
        for lo, hi in zip(BNDS[:-1], BNDS[1:]):
            ta = _bound17v(carry[1], carry[2])
            tb = _bound17v(carry[4], carry[5])
            carry = plsc.parallel_loop(lo, hi, carry=carry,
                                       unroll=2)(chunk(ta, tb))
        cnta, m1a, m2a, cntb, m1b, m2b = carry

        def prep(sdr, sir, cnt, m1, m2, s2dr, s2ir):
            tfin = _bound17v(m1, m2)
            cnt_s = cnt[0]

            def refine(i, cnt2):
                s = pl.ds(i * 16, 16)
                v = sdr[s]
                iv = sir[s]
                posv = i * 16 + lanes
                msk = (v <= tfin) & (posv < cnt)
                pos = cnt2 + plsc.cumsum(msk.astype(jnp.int32)) - 1
                plsc.store_scatter(s2dr, [pos], v, mask=msk)
                plsc.store_scatter(s2ir, [pos], iv, mask=msk)
                return cnt2 + plsc.all_reduce_population_count(msk)

            nch1 = (cnt_s + 15) // 16
            cnt2 = plsc.parallel_loop(
                0, nch1, carry=jnp.zeros((16,), jnp.int32), unroll=2)(refine)
            return cnt2, cnt2[0]

        c2a, c2as = prep(sd, si, cnta, m1a, m2a, s2d, s2i)
        c2b, c2bs = prep(sdB, siB, cntb, m1b, m2b, s2dB, s2iB)

        def regs(s2dr, s2ir, cnt2):
            def masked(j):
                vj = s2dr[pl.ds(j * 16, 16)]
                return jnp.where(j * 16 + lanes < cnt2, vj, _INF)

            return (masked(0), masked(1), masked(2),
                    s2ir[pl.ds(0, 16)], s2ir[pl.ds(16, 16)],
                    s2ir[pl.ds(32, 16)])

        def one_step(k, wins, v1, v2, v3, i1, i2, i3):
            m = _vmins(jnp.minimum(jnp.minimum(v1, v2), v3))
            l1 = plsc.all_reduce_ffs(v1 == m)
            l2 = plsc.all_reduce_ffs(v2 == m)
            l3 = plsc.all_reduce_ffs(v3 == m)
            use1 = l1 < 16
            use2 = (~use1) & (l2 < 16)
            use3 = (~use1) & (~use2)
            g1 = jnp.take_along_axis(i1, jnp.minimum(l1, 15), axis=0)
            g2 = jnp.take_along_axis(i2, jnp.minimum(l2, 15), axis=0)
            g3 = jnp.take_along_axis(i3, jnp.minimum(l3, 15), axis=0)
            wi = jnp.where(use1, g1, jnp.where(use2, g2, g3))
            v1 = jnp.where(use1 & (lanes == l1), _INF, v1)
            v2 = jnp.where(use2 & (lanes == l2), _INF, v2)
            v3 = jnp.where(use3 & (lanes == l3), _INF, v3)
            wins = jnp.where(lanes == k - 1, wi, wins)
            return wins, v1, v2, v3

        def select_fast2(_):
            va = regs(s2d, s2i, c2a)
            vb = regs(s2dB, s2iB, c2b)
            zero = jnp.full((16,), jnp.int32(0))

            def sel(k, carry):
                wa, a1, a2, a3, wb, b1, b2, b3 = carry
                wa, a1, a2, a3 = one_step(k, wa, a1, a2, a3,
                                          va[3], va[4], va[5])
                wb, b1, b2, b3 = one_step(k, wb, b1, b2, b3,
                                          vb[3], vb[4], vb[5])
                return wa, a1, a2, a3, wb, b1, b2, b3

            out = lax.fori_loop(
                0, NSEL, sel,
                (zero, va[0], va[1], va[2], zero, vb[0], vb[1], vb[2]))
            return out[0], out[4]

        def select_slow2(_):
            def one(s2dr, s2ir, cnt2, cnt2_s):
                nch2 = (cnt2_s + 15) // 16

                def sel(k, carry):
                    wins, _ = carry

                    def scan(i, carry):
                        bv, bp = carry
                        s = pl.ds(i * 16, 16)
                        v = s2dr[s]
                        posv = i * 16 + lanes
                        v = jnp.where(posv < cnt2, v, _INF)
                        m = _vmins(v)
                        l = plsc.all_reduce_ffs(v == m)
                        p = i * 16 + jnp.minimum(l, 15)
                        mm = jnp.take_along_axis(v, jnp.minimum(l, 15),
                                                 axis=0)
                        better = mm < bv
                        bv = jnp.where(better, mm, bv)
                        bp = jnp.where(better, p, bp)
                        return bv, bp

                    bv, bp = lax.fori_loop(
                        0, nch2, scan,
                        (jnp.full((16,), _INF),
                         jnp.full((16,), jnp.int32(0))))
                    wi = plsc.load_gather(s2ir, [bp])
                    plsc.store_scatter(s2dr, [bp], jnp.full((16,), _INF),
                                       mask=lanes == 0)
                    wins = jnp.where(lanes == k - 1, wi, wins)
                    return wins, 0

                wins, _ = lax.fori_loop(
                    0, NSEL, sel, (jnp.full((16,), jnp.int32(0)), 0))
                return wins

            return (one(s2d, s2i, c2a, c2as), one(s2dB, s2iB, c2b, c2bs))

        winsA, winsB = lax.cond((c2as <= 48) & (c2bs <= 48),
                                select_fast2, select_slow2, 0)
        ob[pl.ds((pq * 2) * K, K)] = winsA
        ob[pl.ds((pq * 2 + 1) * K, K)] = winsB
        return 0

    lax.fori_loop(0, QPW // 2, per_pair, 0)

    pltpu.sync_copy(ob, out.at[b, pl.ds(qs * K, QPW * K)])


def _round_bf16(x):
    bits = lax.bitcast_convert_type(x, jnp.uint32)
    r = bits + jnp.uint32(0x7FFF) + ((bits >> jnp.uint32(16)) & jnp.uint32(1))
    r = r & jnp.uint32(0xFFFF0000)
    return lax.bitcast_convert_type(r, jnp.float32)


@jax.jit
def _run(xyz2, xyz1):
    x2r = _round_bf16(xyz2)
    x1r = _round_bf16(xyz1)
    ins = (xyz2[:, :, 0], xyz2[:, :, 1], xyz2[:, :, 2],
           x2r[:, :, 0], x2r[:, :, 1], x2r[:, :, 2],
           xyz1[:, :, 0], xyz1[:, :, 1], xyz1[:, :, 2],
           x1r[:, :, 0], x1r[:, :, 1], x1r[:, :, 2])
    mesh = plsc.VectorSubcoreMesh(core_axis_name="c", subcore_axis_name="s")
    f = functools.partial(
        pl.kernel,
        out_type=jax.ShapeDtypeStruct((B, N1 * K), jnp.int32),
        mesh=mesh,
        compiler_params=pltpu.CompilerParams(needs_layout_passes=False),
        scratch_types=[
            pltpu.VMEM((N2,), jnp.float32),
            pltpu.VMEM((N2,), jnp.float32),
            pltpu.VMEM((N2,), jnp.float32),
            pltpu.VMEM((N2,), jnp.float32),
            pltpu.VMEM((N2,), jnp.float32),
            pltpu.VMEM((N2,), jnp.float32),
            pltpu.VMEM((N2,), jnp.float32),
            pltpu.VMEM((QPW,), jnp.float32),
            pltpu.VMEM((QPW,), jnp.float32),
            pltpu.VMEM((QPW,), jnp.float32),
            pltpu.VMEM((QPW,), jnp.float32),
            pltpu.VMEM((QPW,), jnp.float32),
            pltpu.VMEM((QPW,), jnp.float32),
            pltpu.VMEM((QPW,), jnp.float32),
            pltpu.VMEM((N2,), jnp.float32),
            pltpu.VMEM((N2,), jnp.int32),
            pltpu.VMEM((N2,), jnp.float32),
            pltpu.VMEM((N2,), jnp.int32),
            pltpu.VMEM((N2,), jnp.float32),
            pltpu.VMEM((N2,), jnp.int32),
            pltpu.VMEM((N2,), jnp.float32),
            pltpu.VMEM((N2,), jnp.int32),
            pltpu.VMEM((QPW * K,), jnp.int32),
        ],
    )(_sc_body)
    out = f(*ins)
    return out.reshape(B, N1, K)


def kernel(xyz2, xyz1):
    return _run(xyz2, xyz1)


# speedup vs baseline: 1.0145x; 1.0145x over previous
import functools

import jax
import jax.numpy as jnp
from jax import lax
from jax.experimental import pallas as pl
from jax.experimental.pallas import tpu as pltpu
from jax.experimental.pallas import tpu_sc as plsc

B = 4
N1 = 4096
N2 = 4096
K = 16
NSEL = K + 1
QPW = 512
NCH = N2 // 16
WARM = 4
BNDS = (4, 16, 64, 256)

_INF = float(jnp.inf)


def _lanes():
    return lax.broadcasted_iota(jnp.int32, (16,), 0)


def _last15():
    return jnp.full((16,), 15, jnp.int32)


def _vmaxs(v):
    return jnp.take_along_axis(plsc.cummax(v), _last15(), axis=0)


def _vmins(v):
    return -_vmaxs(-v)


def _bound17v(m1, m2):
    a = lax.sort(m1)
    bb = lax.rev(lax.sort(m2), (0,))
    return _vmins(jnp.maximum(a, bb))


def _sc_body(x2x, x2y, x2z, x2xr, x2yr, x2zr,
             x1x, x1y, x1z, x1xr, x1yr, x1zr, out,
             rx, ry, rz, rn, ux, uy, uz,
             qx, qy, qz, qn, vx, vy, vz,
             sd, si, sdB, siB, s2d, s2i, s2dB, s2iB, ob):
    cid = lax.axis_index("c")
    sid = lax.axis_index("s")
    wid = sid * 2 + cid
    b = wid // (N1 // QPW)
    qs = (wid % (N1 // QPW)) * QPW

    pltpu.sync_copy(x2x.at[b], ux)
    pltpu.sync_copy(x2y.at[b], uy)
    pltpu.sync_copy(x2z.at[b], uz)
    pltpu.sync_copy(x2xr.at[b], rx)
    pltpu.sync_copy(x2yr.at[b], ry)
    pltpu.sync_copy(x2zr.at[b], rz)
    pltpu.sync_copy(x1x.at[b, pl.ds(qs, QPW)], vx)
    pltpu.sync_copy(x1y.at[b, pl.ds(qs, QPW)], vy)
    pltpu.sync_copy(x1z.at[b, pl.ds(qs, QPW)], vz)
    pltpu.sync_copy(x1xr.at[b, pl.ds(qs, QPW)], qx)
    pltpu.sync_copy(x1yr.at[b, pl.ds(qs, QPW)], qy)
    pltpu.sync_copy(x1zr.at[b, pl.ds(qs, QPW)], qz)

    lanes = _lanes()

    def ref_pre(i, _):
        s = pl.ds(i * 16, 16)
        rn[s] = (ux[s] * ux[s] + uy[s] * uy[s]) + uz[s] * uz[s]
        return 0

    lax.fori_loop(0, NCH, ref_pre, 0, unroll=4)

    def q_pre(i, _):
        s = pl.ds(i * 16, 16)
        qn[s] = (vx[s] * vx[s] + vy[s] * vy[s]) + vz[s] * vz[s]
        return 0

    lax.fori_loop(0, QPW // 16, q_pre, 0, unroll=4)

    def idx_pre(i, _):
        s = pl.ds(i * 16, 16)
        iv = i * 16 + lanes
        si[s] = iv
        siB[s] = iv
        return 0

    lax.fori_loop(0, WARM, idx_pre, 0, unroll=4)

    def per_pair(pq, _):
        def qload(ref, qsplat):
            return plsc.load_gather(ref, [qsplat])

        qa = jnp.broadcast_to(pq * 2, (16,)).astype(jnp.int32)
        qb = qa + 1
        qxa, qya, qza, qna = (qload(qx, qa), qload(qy, qa),
                              qload(qz, qa), qload(qn, qa))
        qxb, qyb, qzb, qnb = (qload(qx, qb), qload(qy, qb),
                              qload(qz, qb), qload(qn, qb))

        def dist2(i):
            s = pl.ds(i * 16, 16)
            vrx, vry, vrz, vrn = rx[s], ry[s], rz[s], rn[s]

            def one(qxv, qyv, qzv, qnv):
                cross = vrx * qxv + vry * qyv + vrz * qzv
                d = (vrn + qnv) - (cross + cross)
                return d, jnp.maximum(d, jnp.float32(0.0))

            return one(qxa, qya, qza, qna), one(qxb, qyb, qzb, qnb)

        def minmax2(dc, m1, m2):
            hi = jnp.maximum(m1, dc)
            return jnp.minimum(m1, dc), jnp.minimum(m2, hi)

        def warm(i, carry):
            m1a, m2a, m1b, m2b = carry
            (_, dca), (_, dcb) = dist2(i)
            s = pl.ds(i * 16, 16)
            sd[s] = dca
            sdB[s] = dcb
            m1a, m2a = minmax2(dca, m1a, m2a)
            m1b, m2b = minmax2(dcb, m1b, m2b)
            return m1a, m2a, m1b, m2b

        inf16 = jnp.full((16,), _INF)
        m1a, m2a, m1b, m2b = plsc.parallel_loop(
            0, WARM, carry=(inf16, inf16, inf16, inf16), unroll=4)(warm)
        cnt0 = jnp.full((16,), WARM * 16, jnp.int32)

        def chunk(ta, tb):
            def go(i, carry):
                cnta, m1a, m2a, cntb, m1b, m2b = carry
                (da, dca), (db, dcb) = dist2(i)
                idxv = i * 16 + lanes
                mska = da <= ta
                posa = cnta + plsc.cumsum(mska.astype(jnp.int32)) - 1
                plsc.store_scatter(sd, [posa], dca, mask=mska)
                plsc.store_scatter(si, [posa], idxv, mask=mska)
                cnta = cnta + plsc.all_reduce_population_count(mska)
                m1a, m2a = minmax2(dca, m1a, m2a)
                mskb = db <= tb
                posb = cntb + plsc.cumsum(mskb.astype(jnp.int32)) - 1
                plsc.store_scatter(sdB, [posb], dcb, mask=mskb)
                plsc.store_scatter(siB, [posb], idxv, mask=mskb)
                cntb = cntb + plsc.all_reduce_population_count(mskb)
                m1b, m2b = minmax2(dcb, m1b, m2b)
                return cnta, m1a, m2a, cntb, m1b, m2b
            return go

        carry = (cnt0, m1a, m2a, cnt0, m1b, m2b)
        for lo, hi in zip(BNDS[:-1], BNDS[1:]):
            ta = _bound17v(carry[1], carry[2])
            tb = _bound17v(carry[4], carry[5])
            carry = plsc.parallel_loop(lo, hi, carry=carry,
                                       unroll=2)(chunk(ta, tb))
        cnta, m1a, m2a, cntb, m1b, m2b = carry

        def prep(sdr, sir, cnt, m1, m2, s2dr, s2ir):
            tfin = _bound17v(m1, m2)
            cnt_s = cnt[0]

            def refine(i, cnt2):
                s = pl.ds(i * 16, 16)
                v = sdr[s]
                iv = sir[s]
                posv = i * 16 + lanes
                msk = (v <= tfin) & (posv < cnt)
                pos = cnt2 + plsc.cumsum(msk.astype(jnp.int32)) - 1
                plsc.store_scatter(s2dr, [pos], v, mask=msk)
                plsc.store_scatter(s2ir, [pos], iv, mask=msk)
                return cnt2 + plsc.all_reduce_population_count(msk)

            nch1 = (cnt_s + 15) // 16
            cnt2 = plsc.parallel_loop(
                0, nch1, carry=jnp.zeros((16,), jnp.int32), unroll=2)(refine)
            return cnt2, cnt2[0]

        c2a, c2as = prep(sd, si, cnta, m1a, m2a, s2d, s2i)
        c2b, c2bs = prep(sdB, siB, cntb, m1b, m2b, s2dB, s2iB)

        def regs(s2dr, s2ir, cnt2):
            def masked(j):
                vj = s2dr[pl.ds(j * 16, 16)]
                return jnp.where(j * 16 + lanes < cnt2, vj, _INF)

            return (masked(0), masked(1), masked(2),
                    s2ir[pl.ds(0, 16)], s2ir[pl.ds(16, 16)],
                    s2ir[pl.ds(32, 16)])

        def one_step(k, wins, v1, v2, v3, i1, i2, i3):
            m = _vmins(jnp.minimum(jnp.minimum(v1, v2), v3))
            l1 = plsc.all_reduce_ffs(v1 == m)
            l2 = plsc.all_reduce_ffs(v2 == m)
            l3 = plsc.all_reduce_ffs(v3 == m)
            use1 = l1 < 16
            use2 = (~use1) & (l2 < 16)
            use3 = (~use1) & (~use2)
            g1 = jnp.take_along_axis(i1, jnp.minimum(l1, 15), axis=0)
            g2 = jnp.take_along_axis(i2, jnp.minimum(l2, 15), axis=0)
            g3 = jnp.take_along_axis(i3, jnp.minimum(l3, 15), axis=0)
            wi = jnp.where(use1, g1, jnp.where(use2, g2, g3))
            v1 = jnp.where(use1 & (lanes == l1), _INF, v1)
            v2 = jnp.where(use2 & (lanes == l2), _INF, v2)
            v3 = jnp.where(use3 & (lanes == l3), _INF, v3)
            wins = jnp.where(lanes == k - 1, wi, wins)
            return wins, v1, v2, v3

        def select_fast2(_):
            va = regs(s2d, s2i, c2a)
            vb = regs(s2dB, s2iB, c2b)
            zero = jnp.full((16,), jnp.int32(0))

            def sel(k, carry):
                wa, a1, a2, a3, wb, b1, b2, b3 = carry
                wa, a1, a2, a3 = one_step(k, wa, a1, a2, a3,
                                          va[3], va[4], va[5])
                wb, b1, b2, b3 = one_step(k, wb, b1, b2, b3,
                                          vb[3], vb[4], vb[5])
                return wa, a1, a2, a3, wb, b1, b2, b3

            out = lax.fori_loop(
                0, NSEL, sel,
                (zero, va[0], va[1], va[2], zero, vb[0], vb[1], vb[2]))
            return out[0], out[4]

        def select_slow2(_):
            def one(s2dr, s2ir, cnt2, cnt2_s):
                nch2 = (cnt2_s + 15) // 16

                def sel(k, carry):
                    wins, _ = carry

                    def scan(i, carry):
                        bv, bp = carry
                        s = pl.ds(i * 16, 16)
                        v = s2dr[s]
                        posv = i * 16 + lanes
                        v = jnp.where(posv < cnt2, v, _INF)
                        m = _vmins(v)
                        l = plsc.all_reduce_ffs(v == m)
                        p = i * 16 + jnp.minimum(l, 15)
                        mm = jnp.take_along_axis(v, jnp.minimum(l, 15),
                                                 axis=0)
                        better = mm < bv
                        bv = jnp.where(better, mm, bv)
                        bp = jnp.where(better, p, bp)
                        return bv, bp

                    bv, bp = lax.fori_loop(
                        0, nch2, scan,
                        (jnp.full((16,), _INF),
                         jnp.full((16,), jnp.int32(0))))
                    wi = plsc.load_gather(s2ir, [bp])
                    plsc.store_scatter(s2dr, [bp], jnp.full((16,), _INF),
                                       mask=lanes == 0)
                    wins = jnp.where(lanes == k - 1, wi, wins)
                    return wins, 0

                wins, _ = lax.fori_loop(
                    0, NSEL, sel, (jnp.full((16,), jnp.int32(0)), 0))
                return wins

            return (one(s2d, s2i, c2a, c2as), one(s2dB, s2iB, c2b, c2bs))

        winsA, winsB = lax.cond((c2as <= 48) & (c2bs <= 48),
                                select_fast2, select_slow2, 0)
        ob[pl.ds((pq * 2) * K, K)] = winsA
        ob[pl.ds((pq * 2 + 1) * K, K)] = winsB
        return 0

    lax.fori_loop(0, QPW // 2, per_pair, 0)

    pltpu.sync_copy(ob, out.at[b, pl.ds(qs * K, QPW * K)])


def _round_bf16(x):
    bits = lax.bitcast_convert_type(x, jnp.uint32)
    r = bits + jnp.uint32(0x7FFF) + ((bits >> jnp.uint32(16)) & jnp.uint32(1))
    r = r & jnp.uint32(0xFFFF0000)
    return lax.bitcast_convert_type(r, jnp.float32)


@jax.jit
def _run(xyz2, xyz1):
    x2r = _round_bf16(xyz2)
    x1r = _round_bf16(xyz1)
    ins = (xyz2[:, :, 0], xyz2[:, :, 1], xyz2[:, :, 2],
           x2r[:, :, 0], x2r[:, :, 1], x2r[:, :, 2],
           xyz1[:, :, 0], xyz1[:, :, 1], xyz1[:, :, 2],
           x1r[:, :, 0], x1r[:, :, 1], x1r[:, :, 2])
    mesh = plsc.VectorSubcoreMesh(core_axis_name="c", subcore_axis_name="s")
    f = functools.partial(
        pl.kernel,
        out_type=jax.ShapeDtypeStruct((B, N1 * K), jnp.int32),
        mesh=mesh,
        compiler_params=pltpu.CompilerParams(needs_layout_passes=False),
        scratch_types=[
            pltpu.VMEM((N2,), jnp.float32),
            pltpu.VMEM((N2,), jnp.float32),
            pltpu.VMEM((N2,), jnp.float32),
            pltpu.VMEM((N2,), jnp.float32),
            pltpu.VMEM((N2,), jnp.float32),
            pltpu.VMEM((N2,), jnp.float32),
            pltpu.VMEM((N2,), jnp.float32),
            pltpu.VMEM((QPW,), jnp.float32),
            pltpu.VMEM((QPW,), jnp.float32),
            pltpu.VMEM((QPW,), jnp.float32),
            pltpu.VMEM((QPW,), jnp.float32),
            pltpu.VMEM((QPW,), jnp.float32),
            pltpu.VMEM((QPW,), jnp.float32),
            pltpu.VMEM((QPW,), jnp.float32),
            pltpu.VMEM((N2,), jnp.float32),
            pltpu.VMEM((N2,), jnp.int32),
            pltpu.VMEM((N2,), jnp.float32),
            pltpu.VMEM((N2,), jnp.int32),
            pltpu.VMEM((N2,), jnp.float32),
            pltpu.VMEM((N2,), jnp.int32),
            pltpu.VMEM((N2,), jnp.float32),
            pltpu.VMEM((N2,), jnp.int32),
            pltpu.VMEM((QPW * K,), jnp.int32),
        ],
    )(_sc_body)
    out = f(*ins)
    return out.reshape(B, N1, K)


def kernel(xyz2, xyz1):
    return _run(xyz2, xyz1)
